# Initial kernel scaffold; baseline (speedup 1.0000x reference)
import jax
import jax.numpy as jnp
from jax.experimental import pallas as pl


def kernel(query_points, target_points, target_features):
    raise NotImplementedError("write your pallas kernel here")


# speedup vs baseline: 1.4281x; 1.4281x over previous
import functools

import jax
import jax.numpy as jnp
from jax import lax
from jax.experimental import pallas as pl
from jax.experimental.pallas import tpu as pltpu
from jax.experimental.pallas import tpu_sc as plsc

N_Q = 16384
N_T = 8192
D_F = 512


_TQ = 256


_WIN = N_T // 2


def _argmin_body(q_ref, tt_ref, idx_ref):
    q = q_ref[...]
    tt = tt_ref[...]
    qb = q.astype(jnp.bfloat16)
    ttb = tt.astype(jnp.bfloat16)
    qt = lax.dot_general(qb, ttb, (((1,), (0,)), ((), ())),
                         preferred_element_type=jnp.float32)
    qsq = (q[:, 0:1] * q[:, 0:1] + q[:, 2:3] * q[:, 2:3]) + q[:, 1:2] * q[:, 1:2]
    tsq = (tt[0:1, :] * tt[0:1, :] + tt[2:3, :] * tt[2:3, :]) + tt[1:2, :] * tt[1:2, :]
    d = (qsq - 2.0 * qt) + tsq
    d1 = d[:, :_WIN]
    d2 = d[:, _WIN:]
    ii = lax.broadcasted_iota(jnp.int32, d1.shape, 1)
    big = jnp.int32(N_T)
    m1 = jnp.min(d1, axis=-1, keepdims=True)
    i1 = jnp.min(jnp.where(d1 == m1, ii, big), axis=-1)
    m2 = jnp.min(d2, axis=-1, keepdims=True)
    i2 = jnp.min(jnp.where(d2 == m2, ii, big), axis=-1) + _WIN
    m1b = m1.astype(jnp.bfloat16).astype(jnp.float32)
    use2 = (m2 < m1b)[:, 0]
    idx = jnp.where(use2, i2, i1)
    idx_ref[...] = idx[:, None]


_argmin_call = pl.pallas_call(
    _argmin_body,
    grid=(N_Q // _TQ,),
    in_specs=[
        pl.BlockSpec((_TQ, 3), lambda i: (i, 0)),
        pl.BlockSpec((3, N_T), lambda i: (0, 0)),
    ],
    out_specs=pl.BlockSpec((_TQ, 1), lambda i: (i, 0)),
    out_shape=jax.ShapeDtypeStruct((N_Q, 1), jnp.int32),
)


_NC, _NS = 2, 16
_NW = _NC * _NS
_BPW = N_Q // _NW
_CH = 128
_CHUNKS = _BPW // _CH


def _gather_body(table_hbm, idx_hbm, out_hbm, idx_v, rows_v, sem):
    wid = lax.axis_index("s") * _NC + lax.axis_index("c")
    pltpu.sync_copy(idx_hbm.at[wid], idx_v)
    for c in range(_CHUNKS):
        pltpu.async_copy(table_hbm.at[idx_v.at[c]], rows_v, sem).wait()
        pltpu.sync_copy(rows_v, out_hbm.at[pl.ds(wid * _BPW + c * _CH, _CH)])


@functools.cache
def _gather_call():
    return pl.kernel(
        _gather_body,
        out_type=jax.ShapeDtypeStruct((N_Q, D_F), jnp.float32),
        mesh=plsc.VectorSubcoreMesh(core_axis_name="c", subcore_axis_name="s"),
        scratch_types=[
            pltpu.VMEM((_CHUNKS, _CH), jnp.int32),
            pltpu.VMEM((_CH, D_F), jnp.float32),
            pltpu.SemaphoreType.DMA,
        ],
    )


def kernel(query_points, target_points, target_features):
    tt = target_points.T
    idx = _argmin_call(query_points, tt)
    idx3 = idx.reshape(_NW, _CHUNKS, _CH)
    query_features = _gather_call()(target_features, idx3)
    return (query_points, query_features)


# speedup vs baseline: 1.7373x; 1.2165x over previous
import functools

import jax
import jax.numpy as jnp
from jax import lax
from jax.experimental import pallas as pl
from jax.experimental.pallas import tpu as pltpu
from jax.experimental.pallas import tpu_sc as plsc

N_Q = 16384
N_T = 8192
D_F = 512


_TQ = 256


_WIN = N_T // 2
_LANES = 128
_NSLICE = _WIN // _LANES


def _window_argmin(qsq, qtn, tsq, base, lane_iota):
    def dslice(j):
        lo = base + j * _LANES
        return (qsq + qtn[:, lo:lo + _LANES]) + tsq[:, lo:lo + _LANES]

    val = dslice(0)
    idxv = jnp.zeros(val.shape, jnp.float32)
    for j in range(1, _NSLICE):
        dj = dslice(j)
        mask = dj < val
        val = jnp.where(mask, dj, val)
        idxv = jnp.where(mask, jnp.float32(j), idxv)
    gidx = idxv * jnp.float32(_LANES) + lane_iota
    m = jnp.min(val, axis=-1, keepdims=True)
    i = jnp.min(jnp.where(val == m, gidx, jnp.float32(1e9)), axis=-1)
    return m, i


def _argmin_body(q_ref, tt_ref, idx_ref):
    q = q_ref[...]
    tt = tt_ref[...]
    qb = (q * -2.0).astype(jnp.bfloat16)
    ttb = tt.astype(jnp.bfloat16)
    qtn = lax.dot_general(qb, ttb, (((1,), (0,)), ((), ())),
                          preferred_element_type=jnp.float32)
    qsq = (q[:, 0:1] * q[:, 0:1] + q[:, 2:3] * q[:, 2:3]) + q[:, 1:2] * q[:, 1:2]
    tsq = (tt[0:1, :] * tt[0:1, :] + tt[2:3, :] * tt[2:3, :]) + tt[1:2, :] * tt[1:2, :]
    lane_iota = lax.broadcasted_iota(
        jnp.int32, (q.shape[0], _LANES), 1).astype(jnp.float32)
    m1, i1 = _window_argmin(qsq, qtn, tsq, 0, lane_iota)
    m2, i2 = _window_argmin(qsq, qtn, tsq, _WIN, lane_iota)
    m1b = m1.astype(jnp.bfloat16).astype(jnp.float32)
    use2 = (m2 < m1b)[:, 0]
    idx = jnp.where(use2, i2 + jnp.float32(_WIN), i1).astype(jnp.int32)
    idx_ref[...] = idx[:, None]


_argmin_call = pl.pallas_call(
    _argmin_body,
    grid=(N_Q // _TQ,),
    in_specs=[
        pl.BlockSpec((_TQ, 3), lambda i: (i, 0)),
        pl.BlockSpec((3, N_T), lambda i: (0, 0)),
    ],
    out_specs=pl.BlockSpec((_TQ, 1), lambda i: (i, 0)),
    out_shape=jax.ShapeDtypeStruct((N_Q, 1), jnp.int32),
)


_NC, _NS = 2, 16
_NW = _NC * _NS
_BPW = N_Q // _NW
_CH = 128
_CHUNKS = _BPW // _CH


def _gather_body(table_hbm, idx_hbm, out_hbm, idx_v, rows_v, sem):
    wid = lax.axis_index("s") * _NC + lax.axis_index("c")
    pltpu.sync_copy(idx_hbm.at[wid], idx_v)
    for c in range(_CHUNKS):
        pltpu.async_copy(table_hbm.at[idx_v.at[c]], rows_v, sem).wait()
        pltpu.sync_copy(rows_v, out_hbm.at[pl.ds(wid * _BPW + c * _CH, _CH)])


@functools.cache
def _gather_call():
    return pl.kernel(
        _gather_body,
        out_type=jax.ShapeDtypeStruct((N_Q, D_F), jnp.float32),
        mesh=plsc.VectorSubcoreMesh(core_axis_name="c", subcore_axis_name="s"),
        scratch_types=[
            pltpu.VMEM((_CHUNKS, _CH), jnp.int32),
            pltpu.VMEM((_CH, D_F), jnp.float32),
            pltpu.SemaphoreType.DMA,
        ],
    )


def kernel(query_points, target_points, target_features):
    tt = target_points.T
    idx = _argmin_call(query_points, tt)
    idx3 = idx.reshape(_NW, _CHUNKS, _CH)
    query_features = _gather_call()(target_features, idx3)
    return (query_points, query_features)


# speedup vs baseline: 1.8846x; 1.0848x over previous
import functools

import jax
import jax.numpy as jnp
from jax import lax
from jax.experimental import pallas as pl
from jax.experimental.pallas import tpu as pltpu
from jax.experimental.pallas import tpu_sc as plsc

N_Q = 16384
N_T = 8192
D_F = 512


_TQ = 512


_WIN = N_T // 2
_LANES = 128
_NSLICE = _WIN // _LANES


def _window_argmin(qsq, qtn, tsq, base, lane_iota):
    def dslice(j):
        lo = base + j * _LANES
        return (qsq + qtn[:, lo:lo + _LANES]) + tsq[:, lo:lo + _LANES]

    val = dslice(0)
    idxv = jnp.zeros(val.shape, jnp.float32)
    for j in range(1, _NSLICE):
        dj = dslice(j)
        mask = dj < val
        val = jnp.where(mask, dj, val)
        idxv = jnp.where(mask, jnp.float32(j), idxv)
    gidx = idxv * jnp.float32(_LANES) + lane_iota
    m = jnp.min(val, axis=-1, keepdims=True)
    i = jnp.min(jnp.where(val == m, gidx, jnp.float32(1e9)), axis=-1)
    return m, i


def _argmin_body(q_ref, tt_ref, idx_ref):
    q = q_ref[...]
    tt = tt_ref[...]
    qb = (q * -2.0).astype(jnp.bfloat16)
    ttb = tt.astype(jnp.bfloat16)
    qtn = lax.dot_general(qb, ttb, (((1,), (0,)), ((), ())),
                          preferred_element_type=jnp.float32)
    qsq = (q[:, 0:1] * q[:, 0:1] + q[:, 2:3] * q[:, 2:3]) + q[:, 1:2] * q[:, 1:2]
    tsq = (tt[0:1, :] * tt[0:1, :] + tt[2:3, :] * tt[2:3, :]) + tt[1:2, :] * tt[1:2, :]
    lane_iota = lax.broadcasted_iota(
        jnp.int32, (q.shape[0], _LANES), 1).astype(jnp.float32)
    m1, i1 = _window_argmin(qsq, qtn, tsq, 0, lane_iota)
    m2, i2 = _window_argmin(qsq, qtn, tsq, _WIN, lane_iota)
    m1b = m1.astype(jnp.bfloat16).astype(jnp.float32)
    use2 = (m2 < m1b)[:, 0]
    idx = jnp.where(use2, i2 + jnp.float32(_WIN), i1).astype(jnp.int32)
    idx_ref[...] = idx[:, None]


_argmin_call = pl.pallas_call(
    _argmin_body,
    grid=(N_Q // _TQ,),
    in_specs=[
        pl.BlockSpec((_TQ, 3), lambda i: (i, 0)),
        pl.BlockSpec((3, N_T), lambda i: (0, 0)),
    ],
    out_specs=pl.BlockSpec((_TQ, 1), lambda i: (i, 0)),
    out_shape=jax.ShapeDtypeStruct((N_Q, 1), jnp.int32),
)


_NC, _NS = 2, 16
_NW = _NC * _NS
_BPW = N_Q // _NW
_CH = 128
_CHUNKS = _BPW // _CH


def _gather_body(table_hbm, idx_hbm, out_hbm, idx_v, rows_v, sem):
    wid = lax.axis_index("s") * _NC + lax.axis_index("c")
    pltpu.sync_copy(idx_hbm.at[wid], idx_v)
    for c in range(_CHUNKS):
        pltpu.async_copy(table_hbm.at[idx_v.at[c]], rows_v, sem).wait()
        pltpu.sync_copy(rows_v, out_hbm.at[pl.ds(wid * _BPW + c * _CH, _CH)])


@functools.cache
def _gather_call():
    return pl.kernel(
        _gather_body,
        out_type=jax.ShapeDtypeStruct((N_Q, D_F), jnp.float32),
        mesh=plsc.VectorSubcoreMesh(core_axis_name="c", subcore_axis_name="s"),
        scratch_types=[
            pltpu.VMEM((_CHUNKS, _CH), jnp.int32),
            pltpu.VMEM((_CH, D_F), jnp.float32),
            pltpu.SemaphoreType.DMA,
        ],
    )


def kernel(query_points, target_points, target_features):
    tt = target_points.T
    idx = _argmin_call(query_points, tt)
    idx3 = idx.reshape(_NW, _CHUNKS, _CH)
    query_features = _gather_call()(target_features, idx3)
    return (query_points, query_features)


# speedup vs baseline: 1.9824x; 1.0519x over previous
import functools

import jax
import jax.numpy as jnp
from jax import lax
from jax.experimental import pallas as pl
from jax.experimental.pallas import tpu as pltpu
from jax.experimental.pallas import tpu_sc as plsc

N_Q = 16384
N_T = 8192
D_F = 512


_TQ = 1024


_WIN = N_T // 2
_LANES = 128
_NSLICE = _WIN // _LANES


def _window_argmin(qsq, qtn, tsq, base, lane_iota):
    def dslice(j):
        lo = base + j * _LANES
        return (qsq + qtn[:, lo:lo + _LANES]) + tsq[:, lo:lo + _LANES]

    val = dslice(0)
    idxv = jnp.zeros(val.shape, jnp.float32)
    for j in range(1, _NSLICE):
        dj = dslice(j)
        mask = dj < val
        val = jnp.where(mask, dj, val)
        idxv = jnp.where(mask, jnp.float32(j), idxv)
    gidx = idxv * jnp.float32(_LANES) + lane_iota
    m = jnp.min(val, axis=-1, keepdims=True)
    i = jnp.min(jnp.where(val == m, gidx, jnp.float32(1e9)), axis=-1)
    return m, i


def _argmin_body(q_ref, tt_ref, idx_ref):
    q = q_ref[...]
    tt = tt_ref[...]
    qb = (q * -2.0).astype(jnp.bfloat16)
    ttb = tt.astype(jnp.bfloat16)
    qtn = lax.dot_general(qb, ttb, (((1,), (0,)), ((), ())),
                          preferred_element_type=jnp.float32)
    qsq = (q[:, 0:1] * q[:, 0:1] + q[:, 2:3] * q[:, 2:3]) + q[:, 1:2] * q[:, 1:2]
    tsq = (tt[0:1, :] * tt[0:1, :] + tt[2:3, :] * tt[2:3, :]) + tt[1:2, :] * tt[1:2, :]
    lane_iota = lax.broadcasted_iota(
        jnp.int32, (q.shape[0], _LANES), 1).astype(jnp.float32)
    m1, i1 = _window_argmin(qsq, qtn, tsq, 0, lane_iota)
    m2, i2 = _window_argmin(qsq, qtn, tsq, _WIN, lane_iota)
    m1b = m1.astype(jnp.bfloat16).astype(jnp.float32)
    use2 = (m2 < m1b)[:, 0]
    idx = jnp.where(use2, i2 + jnp.float32(_WIN), i1).astype(jnp.int32)
    idx_ref[...] = idx[:, None]


_argmin_call = pl.pallas_call(
    _argmin_body,
    grid=(N_Q // _TQ,),
    in_specs=[
        pl.BlockSpec((_TQ, 3), lambda i: (i, 0)),
        pl.BlockSpec((3, N_T), lambda i: (0, 0)),
    ],
    out_specs=pl.BlockSpec((_TQ, 1), lambda i: (i, 0)),
    out_shape=jax.ShapeDtypeStruct((N_Q, 1), jnp.int32),
)


_NC, _NS = 2, 16
_NW = _NC * _NS
_BPW = N_Q // _NW
_CH = 128
_CHUNKS = _BPW // _CH


def _gather_body(table_hbm, idx_hbm, out_hbm, idx_v, rows_v, sem):
    wid = lax.axis_index("s") * _NC + lax.axis_index("c")
    pltpu.sync_copy(idx_hbm.at[wid], idx_v)
    for c in range(_CHUNKS):
        pltpu.async_copy(table_hbm.at[idx_v.at[c]], rows_v, sem).wait()
        pltpu.sync_copy(rows_v, out_hbm.at[pl.ds(wid * _BPW + c * _CH, _CH)])


@functools.cache
def _gather_call():
    return pl.kernel(
        _gather_body,
        out_type=jax.ShapeDtypeStruct((N_Q, D_F), jnp.float32),
        mesh=plsc.VectorSubcoreMesh(core_axis_name="c", subcore_axis_name="s"),
        scratch_types=[
            pltpu.VMEM((_CHUNKS, _CH), jnp.int32),
            pltpu.VMEM((_CH, D_F), jnp.float32),
            pltpu.SemaphoreType.DMA,
        ],
    )


def kernel(query_points, target_points, target_features):
    tt = target_points.T
    idx = _argmin_call(query_points, tt)
    idx3 = idx.reshape(_NW, _CHUNKS, _CH)
    query_features = _gather_call()(target_features, idx3)
    return (query_points, query_features)


# speedup vs baseline: 1.9987x; 1.0082x over previous
import functools

import jax
import jax.numpy as jnp
from jax import lax
from jax.experimental import pallas as pl
from jax.experimental.pallas import tpu as pltpu
from jax.experimental.pallas import tpu_sc as plsc

N_Q = 16384
N_T = 8192
D_F = 512


_TQ = 2048


_WIN = N_T // 2
_LANES = 128
_NSLICE = _WIN // _LANES


def _window_argmin(qsq, qtn, tsq, base, lane_iota):
    def dslice(j):
        lo = base + j * _LANES
        return (qsq + qtn[:, lo:lo + _LANES]) + tsq[:, lo:lo + _LANES]

    val = dslice(0)
    idxv = jnp.zeros(val.shape, jnp.float32)
    for j in range(1, _NSLICE):
        dj = dslice(j)
        mask = dj < val
        val = jnp.where(mask, dj, val)
        idxv = jnp.where(mask, jnp.float32(j), idxv)
    gidx = idxv * jnp.float32(_LANES) + lane_iota
    m = jnp.min(val, axis=-1, keepdims=True)
    i = jnp.min(jnp.where(val == m, gidx, jnp.float32(1e9)), axis=-1)
    return m, i


def _argmin_body(q_ref, tt_ref, idx_ref):
    q = q_ref[...]
    tt = tt_ref[...]
    qb = (q * -2.0).astype(jnp.bfloat16)
    ttb = tt.astype(jnp.bfloat16)
    qtn = lax.dot_general(qb, ttb, (((1,), (0,)), ((), ())),
                          preferred_element_type=jnp.float32)
    qsq = (q[:, 0:1] * q[:, 0:1] + q[:, 2:3] * q[:, 2:3]) + q[:, 1:2] * q[:, 1:2]
    tsq = (tt[0:1, :] * tt[0:1, :] + tt[2:3, :] * tt[2:3, :]) + tt[1:2, :] * tt[1:2, :]
    lane_iota = lax.broadcasted_iota(
        jnp.int32, (q.shape[0], _LANES), 1).astype(jnp.float32)
    m1, i1 = _window_argmin(qsq, qtn, tsq, 0, lane_iota)
    m2, i2 = _window_argmin(qsq, qtn, tsq, _WIN, lane_iota)
    m1b = m1.astype(jnp.bfloat16).astype(jnp.float32)
    use2 = (m2 < m1b)[:, 0]
    idx = jnp.where(use2, i2 + jnp.float32(_WIN), i1).astype(jnp.int32)
    idx_ref[...] = idx[:, None]


_argmin_call = pl.pallas_call(
    _argmin_body,
    grid=(N_Q // _TQ,),
    in_specs=[
        pl.BlockSpec((_TQ, 3), lambda i: (i, 0)),
        pl.BlockSpec((3, N_T), lambda i: (0, 0)),
    ],
    out_specs=pl.BlockSpec((_TQ, 1), lambda i: (i, 0)),
    out_shape=jax.ShapeDtypeStruct((N_Q, 1), jnp.int32),
)


_NC, _NS = 2, 16
_NW = _NC * _NS
_BPW = N_Q // _NW
_CH = 128
_CHUNKS = _BPW // _CH


def _gather_body(table_hbm, idx_hbm, out_hbm, idx_v, rows_v, sem):
    wid = lax.axis_index("s") * _NC + lax.axis_index("c")
    pltpu.sync_copy(idx_hbm.at[wid], idx_v)
    for c in range(_CHUNKS):
        pltpu.async_copy(table_hbm.at[idx_v.at[c]], rows_v, sem).wait()
        pltpu.sync_copy(rows_v, out_hbm.at[pl.ds(wid * _BPW + c * _CH, _CH)])


@functools.cache
def _gather_call():
    return pl.kernel(
        _gather_body,
        out_type=jax.ShapeDtypeStruct((N_Q, D_F), jnp.float32),
        mesh=plsc.VectorSubcoreMesh(core_axis_name="c", subcore_axis_name="s"),
        scratch_types=[
            pltpu.VMEM((_CHUNKS, _CH), jnp.int32),
            pltpu.VMEM((_CH, D_F), jnp.float32),
            pltpu.SemaphoreType.DMA,
        ],
    )


def kernel(query_points, target_points, target_features):
    tt = target_points.T
    idx = _argmin_call(query_points, tt)
    idx3 = idx.reshape(_NW, _CHUNKS, _CH)
    query_features = _gather_call()(target_features, idx3)
    return (query_points, query_features)
